# baseline (speedup 1.0000x reference)
import jax
import jax.numpy as jnp
from jax import lax
from jax.experimental import pallas as pl
from jax.experimental.pallas import tpu as pltpu
from jax.experimental.pallas import tpu_sc as plsc

_PROBE = 0
D_MODEL = 64
PADL = 129
NUM_WORKERS = 32
LANE = 128
KSUB = 2
GTOK = KSUB * LANE
N_T = 50
N_BTILE = 128
GROUPS = N_T * N_BTILE // KSUB
GPW = GROUPS // NUM_WORKERS


def _body(loc_hbm, table_hbm, out_hbm, idx_v, g0, g1, t0, t1, gs0, gs1,
          ss0, ss1):
    nc = 2
    wid = lax.axis_index("s") * nc + lax.axis_index("c")
    u0 = wid * GPW
    pltpu.sync_copy(loc_hbm.at[pl.ds(u0, GPW)], idx_v)

    gbuf = (g0, g1)
    tbuf = (t0, t1)
    gs = (gs0, gs1)
    ss = (ss0, ss1)
    iota = lax.iota(jnp.int32, 16)

    def fire_gather(u, p):
        pltpu.async_copy(table_hbm.at[idx_v.at[u]], gbuf[p], gs[p])

    def wait_gather(p):
        pltpu.make_async_copy(table_hbm.at[idx_v.at[0]], gbuf[p],
                              gs[p]).wait()

    def transpose_group(p):
        @plsc.parallel_loop(0, GTOK, unroll=4)
        def _(l):
            kvec = jnp.full((16,), l // LANE, jnp.int32)
            lvec = jnp.full((16,), l % LANE, jnp.int32)
            for f0 in range(0, D_MODEL, 16):
                vec = gbuf[p][l, pl.ds(f0, 16)]
                plsc.store_scatter(tbuf[p], [kvec, iota + f0, lvec], vec)

    def fire_stores(u, p):
        c = (u0 + u) * KSUB
        t = c // N_BTILE
        k = c % N_BTILE
        for ksub in range(KSUB):
            for g in range(8):
                pltpu.async_copy(
                    tbuf[p].at[ksub, pl.ds(8 * g, 8), pl.ds(0, LANE)],
                    out_hbm.at[t, g, k + ksub], ss[p])

    def wait_stores(p):
        for _ in range(KSUB * 8):
            pltpu.make_async_copy(
                tbuf[p].at[0, pl.ds(0, 8), pl.ds(0, LANE)],
                out_hbm.at[0, 0, 0], ss[p]).wait()

    fire_gather(0, 0)

    def it_body(v, _):
        for j in (0, 1):
            u = 2 * v + j
            if j == 0:
                fire_gather(u + 1, 1)
            else:
                @pl.when(v < GPW // 2 - 1)
                def _():
                    fire_gather(u + 1, 0)
            wait_gather(j)

            if _PROBE < 2:
                @pl.when(v > 0)
                def _():
                    wait_stores(j)

            if _PROBE == 0:
                transpose_group(j)
            if _PROBE < 2:
                fire_stores(u, j)
        return 0

    lax.fori_loop(0, GPW // 2, it_body, 0)
    if _PROBE < 2:
        wait_stores(0)
        wait_stores(1)


def kernel(locations, table):
    b, t = locations.shape
    loc_groups = locations.T.reshape(GROUPS, GTOK).astype(jnp.int32)
    tab_pad = jnp.pad(table, ((0, 0), (0, 128 - D_MODEL)))

    mesh = plsc.VectorSubcoreMesh(core_axis_name="c", subcore_axis_name="s")
    run = pl.kernel(
        _body,
        mesh=mesh,
        out_type=jax.ShapeDtypeStruct((N_T, 8, N_BTILE, 8, LANE),
                                      jnp.float32),
        scratch_types=[
            pltpu.VMEM((GPW, GTOK), jnp.int32),
            pltpu.VMEM((GTOK, 128), jnp.float32),
            pltpu.VMEM((GTOK, 128), jnp.float32),
            pltpu.VMEM((KSUB, D_MODEL, PADL), jnp.float32),
            pltpu.VMEM((KSUB, D_MODEL, PADL), jnp.float32),
            pltpu.SemaphoreType.DMA,
            pltpu.SemaphoreType.DMA,
            pltpu.SemaphoreType.DMA,
            pltpu.SemaphoreType.DMA,
        ],
        compiler_params=pltpu.CompilerParams(use_tc_tiling_on_sc=False,
                                             needs_layout_passes=False),
    )
    p5 = run(loc_groups, tab_pad)
    out = p5.transpose(2, 4, 0, 1, 3).reshape(b, t, D_MODEL)
    return out


# speedup vs baseline: 1.0906x; 1.0906x over previous
import jax
import jax.numpy as jnp
from jax import lax
from jax.experimental import pallas as pl
from jax.experimental.pallas import tpu as pltpu
from jax.experimental.pallas import tpu_sc as plsc

_PROBE = 0
D_MODEL = 64
PADL = 129
NUM_WORKERS = 32
LANE = 128
KSUB = 2
GTOK = KSUB * LANE
N_T = 50
N_BTILE = 128
GROUPS = N_T * N_BTILE // KSUB
GPW = GROUPS // NUM_WORKERS


def _body(loc_hbm, table_hbm, out_hbm, idx_v, g0, g1, t0, t1, gs0, gs1,
          ss0, ss1):
    nc = 2
    wid = lax.axis_index("s") * nc + lax.axis_index("c")
    u0 = wid * GPW
    pltpu.sync_copy(loc_hbm.at[pl.ds(u0, GPW)], idx_v)

    gbuf = (g0, g1)
    tbuf = (t0, t1)
    gs = (gs0, gs1)
    ss = (ss0, ss1)
    iota = lax.iota(jnp.int32, 16)

    def fire_gather(u, p):
        pltpu.async_copy(table_hbm.at[idx_v.at[u]], gbuf[p], gs[p])

    def wait_gather(p):
        pltpu.make_async_copy(table_hbm.at[idx_v.at[0]], gbuf[p],
                              gs[p]).wait()

    def transpose_group(p):
        @plsc.parallel_loop(0, GTOK, unroll=4)
        def _(l):
            kvec = jnp.full((16,), l // LANE, jnp.int32)
            lvec = jnp.full((16,), l % LANE, jnp.int32)
            for f0 in range(0, D_MODEL, 16):
                vec = gbuf[p][l, pl.ds(f0, 16)]
                plsc.store_scatter(tbuf[p], [kvec, iota + f0, lvec], vec)

    def fire_stores(u, p):
        c = (u0 + u) * KSUB
        t = c // N_BTILE
        k = c % N_BTILE
        for ksub in range(KSUB):
            for g in range(8):
                pltpu.async_copy(
                    tbuf[p].at[ksub, pl.ds(8 * g, 8), pl.ds(0, LANE)],
                    out_hbm.at[t, g, k + ksub], ss[p])

    def wait_stores(p):
        for _ in range(KSUB * 8):
            pltpu.make_async_copy(
                tbuf[p].at[0, pl.ds(0, 8), pl.ds(0, LANE)],
                out_hbm.at[0, 0, 0], ss[p]).wait()

    fire_gather(0, 0)

    def it_body(v, _):
        for j in (0, 1):
            u = 2 * v + j
            if j == 0:
                fire_gather(u + 1, 1)
            else:
                @pl.when(v < GPW // 2 - 1)
                def _():
                    fire_gather(u + 1, 0)
            wait_gather(j)

            if _PROBE < 2:
                @pl.when(v > 0)
                def _():
                    wait_stores(j)

            if _PROBE == 0:
                transpose_group(j)
            if _PROBE < 2:
                fire_stores(u, j)
        return 0

    lax.fori_loop(0, GPW // 2, it_body, 0)
    if _PROBE < 2:
        wait_stores(0)
        wait_stores(1)


_TC_CHUNK = 2048


def _tc_transpose_pad(table):
    n = table.shape[0]

    def body(in_ref, out_ref):
        x = in_ref[...].T
        out_ref[...] = jnp.pad(x, ((0, 0), (0, 128 - D_MODEL)))

    return pl.pallas_call(
        body,
        grid=(pl.cdiv(n, _TC_CHUNK),),
        in_specs=[pl.BlockSpec((D_MODEL, _TC_CHUNK), lambda r: (0, r))],
        out_specs=pl.BlockSpec((_TC_CHUNK, 128), lambda r: (r, 0)),
        out_shape=jax.ShapeDtypeStruct((n, 128), jnp.float32),
    )(table.T)


def kernel(locations, table):
    b, t = locations.shape
    loc_groups = locations.T.reshape(GROUPS, GTOK).astype(jnp.int32)
    tab_pad = _tc_transpose_pad(table)

    mesh = plsc.VectorSubcoreMesh(core_axis_name="c", subcore_axis_name="s")
    run = pl.kernel(
        _body,
        mesh=mesh,
        out_type=jax.ShapeDtypeStruct((N_T, 8, N_BTILE, 8, LANE),
                                      jnp.float32),
        scratch_types=[
            pltpu.VMEM((GPW, GTOK), jnp.int32),
            pltpu.VMEM((GTOK, 128), jnp.float32),
            pltpu.VMEM((GTOK, 128), jnp.float32),
            pltpu.VMEM((KSUB, D_MODEL, PADL), jnp.float32),
            pltpu.VMEM((KSUB, D_MODEL, PADL), jnp.float32),
            pltpu.SemaphoreType.DMA,
            pltpu.SemaphoreType.DMA,
            pltpu.SemaphoreType.DMA,
            pltpu.SemaphoreType.DMA,
        ],
        compiler_params=pltpu.CompilerParams(use_tc_tiling_on_sc=False,
                                             needs_layout_passes=False),
    )
    p5 = run(loc_groups, tab_pad)
    out = p5.transpose(2, 4, 0, 1, 3).reshape(b, t, D_MODEL)
    return out


# speedup vs baseline: 1.0968x; 1.0057x over previous
import jax
import jax.numpy as jnp
from jax import lax
from jax.experimental import pallas as pl
from jax.experimental.pallas import tpu as pltpu
from jax.experimental.pallas import tpu_sc as plsc

D_MODEL = 64
PADL = 129
NUM_WORKERS = 32
LANE = 128
KSUB = 2
GTOK = KSUB * LANE
N_T = 50
N_BTILE = 128
GROUPS = N_T * N_BTILE // KSUB
GPW = GROUPS // NUM_WORKERS


def _body(loc_hbm, table_hbm, out_hbm, idx_v, g0, g1, t0, t1, gs0, gs1,
          ss0, ss1):
    nc = 2
    wid = lax.axis_index("s") * nc + lax.axis_index("c")
    u0 = wid * GPW
    pltpu.sync_copy(loc_hbm.at[pl.ds(u0, GPW)], idx_v)

    gbuf = (g0, g1)
    tbuf = (t0, t1)
    gs = (gs0, gs1)
    ss = (ss0, ss1)
    iota = lax.iota(jnp.int32, 16)

    def fire_gather(u, p):
        pltpu.async_copy(table_hbm.at[idx_v.at[u]], gbuf[p], gs[p])

    def wait_gather(p):
        pltpu.make_async_copy(table_hbm.at[idx_v.at[0]], gbuf[p],
                              gs[p]).wait()

    def transpose_group(p):
        @plsc.parallel_loop(0, GTOK, unroll=4)
        def _(l):
            kvec = jnp.full((16,), l // LANE, jnp.int32)
            lvec = jnp.full((16,), l % LANE, jnp.int32)
            for f0 in range(0, D_MODEL, 16):
                vec = gbuf[p][l, pl.ds(f0, 16)]
                plsc.store_scatter(tbuf[p], [kvec, iota + f0, lvec], vec)

    def fire_stores(u, p):
        c = (u0 + u) * KSUB
        t = c // N_BTILE
        k = c % N_BTILE
        for ksub in range(KSUB):
            for g in range(8):
                pltpu.async_copy(
                    tbuf[p].at[ksub, pl.ds(8 * g, 8), pl.ds(0, LANE)],
                    out_hbm.at[t, g, k + ksub], ss[p])

    def wait_stores(p):
        for _ in range(KSUB * 8):
            pltpu.make_async_copy(
                tbuf[p].at[0, pl.ds(0, 8), pl.ds(0, LANE)],
                out_hbm.at[0, 0, 0], ss[p]).wait()

    fire_gather(0, 0)

    def it_body(v, _):
        for j in (0, 1):
            u = 2 * v + j
            if j == 0:
                fire_gather(u + 1, 1)
            else:
                @pl.when(v < GPW // 2 - 1)
                def _():
                    fire_gather(u + 1, 0)
            wait_gather(j)

            @pl.when(v > 0)
            def _():
                wait_stores(j)

            transpose_group(j)
            fire_stores(u, j)
        return 0

    lax.fori_loop(0, GPW // 2, it_body, 0)
    wait_stores(0)
    wait_stores(1)


_TC_CHUNK = 2048


def _tc_transpose_pad(table):
    n = table.shape[0]

    def body(in_ref, out_ref):
        x = in_ref[...].T
        out_ref[...] = jnp.pad(x, ((0, 0), (0, 128 - D_MODEL)))

    return pl.pallas_call(
        body,
        grid=(pl.cdiv(n, _TC_CHUNK),),
        in_specs=[pl.BlockSpec((D_MODEL, _TC_CHUNK), lambda r: (0, r))],
        out_specs=pl.BlockSpec((_TC_CHUNK, 128), lambda r: (r, 0)),
        out_shape=jax.ShapeDtypeStruct((n, 128), jnp.float32),
    )(table.T)


def kernel(locations, table):
    b, t = locations.shape
    loc_groups = locations.T.reshape(GROUPS, GTOK).astype(jnp.int32)
    tab_pad = _tc_transpose_pad(table)

    mesh = plsc.VectorSubcoreMesh(core_axis_name="c", subcore_axis_name="s")
    run = pl.kernel(
        _body,
        mesh=mesh,
        out_type=jax.ShapeDtypeStruct((N_T, 8, N_BTILE, 8, LANE),
                                      jnp.float32),
        scratch_types=[
            pltpu.VMEM((GPW, GTOK), jnp.int32),
            pltpu.VMEM((GTOK, 128), jnp.float32),
            pltpu.VMEM((GTOK, 128), jnp.float32),
            pltpu.VMEM((KSUB, D_MODEL, PADL), jnp.float32),
            pltpu.VMEM((KSUB, D_MODEL, PADL), jnp.float32),
            pltpu.SemaphoreType.DMA,
            pltpu.SemaphoreType.DMA,
            pltpu.SemaphoreType.DMA,
            pltpu.SemaphoreType.DMA,
        ],
        compiler_params=pltpu.CompilerParams(use_tc_tiling_on_sc=False,
                                             needs_layout_passes=False),
    )
    p5 = run(loc_groups, tab_pad)
    out = p5.transpose(2, 4, 0, 1, 3).reshape(b, t, D_MODEL)
    return out
